# baseline (speedup 1.0000x reference)
import functools

import jax
import jax.numpy as jnp
from jax import lax
from jax.experimental import pallas as pl
from jax.experimental.pallas import tpu as pltpu
from jax.experimental.pallas import tpu_sc as plsc


def _sc_fused_gather(X_domains, ah, at, table):
    info = plsc.get_sparse_core_info()
    nc, ns = info.num_cores, info.num_subcores
    nw = nc * ns
    B = ah.shape[0]
    D = table.shape[1]
    bpw = B // nw
    mesh = plsc.VectorSubcoreMesh(core_axis_name="c", subcore_axis_name="s",
                                  num_cores=nc)

    @functools.partial(
        pl.kernel,
        out_type=(jax.ShapeDtypeStruct((B, D), jnp.float32),
                  jax.ShapeDtypeStruct((B, D), jnp.float32)),
        mesh=mesh,
        scratch_types=[
            pltpu.VMEM((bpw,), jnp.int32),
            pltpu.VMEM((bpw,), jnp.int32),
            pltpu.VMEM((bpw,), jnp.int32),
            pltpu.VMEM((bpw,), jnp.int32),
            pltpu.VMEM((bpw, D), jnp.float32),
            pltpu.VMEM((bpw, D), jnp.float32),
            pltpu.SemaphoreType.DMA,
            pltpu.SemaphoreType.DMA,
        ],
        compiler_params=pltpu.CompilerParams(use_tc_tiling_on_sc=False),
    )
    def gather_kernel(xdom, ah_hbm, at_hbm, tab, eh_out, et_out,
                      ah_v, at_v, ih_v, it_v, eh_v, et_v, sem_h, sem_t):
        wid = lax.axis_index("s") * nc + lax.axis_index("c")
        base = wid * bpw
        pltpu.sync_copy(ah_hbm.at[pl.ds(base, bpw)], ah_v)
        pltpu.sync_copy(at_hbm.at[pl.ds(base, bpw)], at_v)
        ch = pltpu.async_copy(xdom.at[ah_v], ih_v, sem_h)
        ct = pltpu.async_copy(xdom.at[at_v], it_v, sem_t)
        ch.wait()
        gh = pltpu.async_copy(tab.at[ih_v], eh_v, sem_h)
        ct.wait()
        gt = pltpu.async_copy(tab.at[it_v], et_v, sem_t)
        gh.wait()
        pltpu.sync_copy(eh_v, eh_out.at[pl.ds(base, bpw)])
        gt.wait()
        pltpu.sync_copy(et_v, et_out.at[pl.ds(base, bpw)])

    return gather_kernel(X_domains, ah, at, table)


def _mm_body(eh_ref, et_ref, w0_ref, w1_ref, w2_ref, b_ref, o_ref):
    eh = eh_ref[...]
    et = et_ref[...]
    hp = jax.lax.Precision.HIGHEST
    acc = jnp.dot(eh, w0_ref[...], precision=hp, preferred_element_type=jnp.float32)
    acc = acc + jnp.dot(et, w1_ref[...], precision=hp, preferred_element_type=jnp.float32)
    acc = acc + jnp.dot(eh * et, w2_ref[...], precision=hp, preferred_element_type=jnp.float32)
    o_ref[...] = jnp.tanh(acc + b_ref[...])


def _tc_embed(eh, et, W, b):
    B, D = eh.shape
    A = W.shape[1]
    w0, w1, w2 = W[:D], W[D:2 * D], W[2 * D:]
    b2 = b.reshape(1, A)
    blk = 2048
    return pl.pallas_call(
        _mm_body,
        grid=(B // blk,),
        in_specs=[
            pl.BlockSpec((blk, D), lambda i: (i, 0)),
            pl.BlockSpec((blk, D), lambda i: (i, 0)),
            pl.BlockSpec((D, A), lambda i: (0, 0)),
            pl.BlockSpec((D, A), lambda i: (0, 0)),
            pl.BlockSpec((D, A), lambda i: (0, 0)),
            pl.BlockSpec((1, A), lambda i: (0, 0)),
        ],
        out_specs=pl.BlockSpec((blk, A), lambda i: (i, 0)),
        out_shape=jax.ShapeDtypeStruct((B, A), jnp.float32),
    )(eh, et, w0, w1, w2, b2)


def kernel(X_domains, A_predicates, table, W, b):
    ah = A_predicates[:, 0]
    at = A_predicates[:, 1]
    eh, et = _sc_fused_gather(X_domains, ah, at, table)
    return _tc_embed(eh, et, W, b)


# speedup vs baseline: 1.0226x; 1.0226x over previous
import functools

import jax
import jax.numpy as jnp
from jax import lax
from jax.experimental import pallas as pl
from jax.experimental.pallas import tpu as pltpu
from jax.experimental.pallas import tpu_sc as plsc


def _sc_fused_gather(X_domains, a_flat, table):
    info = plsc.get_sparse_core_info()
    nc, ns = info.num_cores, info.num_subcores
    nw = nc * ns
    n = a_flat.shape[0]
    D = table.shape[1]
    npw = n // nw
    mesh = plsc.VectorSubcoreMesh(core_axis_name="c", subcore_axis_name="s",
                                  num_cores=nc)

    @functools.partial(
        pl.kernel,
        out_type=jax.ShapeDtypeStruct((n, D), jnp.float32),
        mesh=mesh,
        scratch_types=[
            pltpu.VMEM((npw,), jnp.int32),
            pltpu.VMEM((npw,), jnp.int32),
            pltpu.VMEM((npw, D), jnp.float32),
            pltpu.SemaphoreType.DMA,
        ],
        compiler_params=pltpu.CompilerParams(use_tc_tiling_on_sc=False),
    )
    def gather_kernel(xdom, a_hbm, tab, rows_out, a_v, ci_v, rows_v, sem):
        wid = lax.axis_index("s") * nc + lax.axis_index("c")
        base = wid * npw
        pltpu.sync_copy(a_hbm.at[pl.ds(base, npw)], a_v)
        pltpu.async_copy(xdom.at[a_v], ci_v, sem).wait()
        pltpu.async_copy(tab.at[ci_v], rows_v, sem).wait()
        pltpu.sync_copy(rows_v, rows_out.at[pl.ds(base, npw)])

    return gather_kernel(X_domains, a_flat, table)


def _mm_body(x_ref, w_ref, b_ref, o_ref):
    x = x_ref[...]
    D = x.shape[1] // 2
    prod = x[:, :D] * x[:, D:]
    hp = jax.lax.Precision.HIGHEST
    acc = jnp.dot(x, w_ref[0:2 * D, :], precision=hp,
                  preferred_element_type=jnp.float32)
    acc = acc + jnp.dot(prod, w_ref[2 * D:3 * D, :], precision=hp,
                        preferred_element_type=jnp.float32)
    o_ref[...] = jnp.tanh(acc + b_ref[...])


def _tc_embed(packed, W, b):
    B, D2 = packed.shape
    K, A = W.shape
    blk = 2048
    return pl.pallas_call(
        _mm_body,
        grid=(B // blk,),
        in_specs=[
            pl.BlockSpec((blk, D2), lambda i: (i, 0)),
            pl.BlockSpec((K, A), lambda i: (0, 0)),
            pl.BlockSpec((A,), lambda i: (0,)),
        ],
        out_specs=pl.BlockSpec((blk, A), lambda i: (i, 0)),
        out_shape=jax.ShapeDtypeStruct((B, A), jnp.float32),
    )(packed, W, b)


def kernel(X_domains, A_predicates, table, W, b):
    B, arity = A_predicates.shape
    D = table.shape[1]
    a_flat = A_predicates.reshape(B * arity)
    rows = _sc_fused_gather(X_domains, a_flat, table)
    packed = rows.reshape(B, arity * D)
    return _tc_embed(packed, W, b)
